# baseline (device time: 100488 ns/iter reference)
import jax
import jax.numpy as jnp
from jax import lax
from jax.experimental import pallas as pl
from jax.experimental.pallas import tpu as pltpu

N_DEV = 8
W_CHUNK = 512
X_CHUNK = 1024


def kernel(x, w_mat):
    m_per, k = x.shape
    _, n = w_mat.shape
    n_per = n // N_DEV
    n_wc = n // W_CHUNK
    n_xc = k // X_CHUNK
    per_dest = n_per // W_CHUNK

    def body(x_hbm, w_hbm, out_ref, xs_ref, xb_ref, wbuf, yb_ref, ybr_ref,
             amax_ref, x_sem, w_sems, own_sem, amax_send, amax_recv,
             ch_send, ch_recv):
        my = lax.axis_index("i")

        def seq_off(i):
            d = 1 + i // per_dest
            t = lax.rem(my + d, N_DEV)
            return t * n_per + (i % per_dest) * W_CHUNK

        def w_copy(i, slot):
            return pltpu.make_async_copy(
                w_hbm.at[:, pl.ds(seq_off(i), W_CHUNK)], wbuf.at[slot],
                w_sems.at[slot])

        w_copy(0, 0).start()
        w_copy(1, 1).start()

        for c in range(n_xc):
            cp = pltpu.make_async_copy(
                x_hbm.at[:, pl.ds(c * X_CHUNK, X_CHUNK)], xs_ref, x_sem)
            cp.start()
            cp.wait()
            xb_ref[:, pl.ds(c * X_CHUNK, X_CHUNK)] = (
                xs_ref[:, :].astype(jnp.bfloat16))

        local_amax = jnp.float32(0.0)
        for i in range(n_wc):
            slot = i % 3
            if i + 2 < n_wc:
                w_copy(i + 2, (i + 2) % 3).start()
            w_copy(i, slot).wait()
            yj = jnp.dot(xb_ref[:, :], wbuf[slot].astype(jnp.bfloat16),
                         preferred_element_type=jnp.float32)
            d = 1 + i // per_dest
            t = lax.rem(my + d, N_DEV)
            c = i % per_dest
            yb_ref[t, :, pl.ds(c * W_CHUNK, W_CHUNK)] = (
                yj.astype(jnp.bfloat16))
            local_amax = jnp.maximum(local_amax, jnp.max(jnp.abs(yj)))
            if c == per_dest - 1:
                if d < N_DEV:
                    pltpu.make_async_remote_copy(
                        src_ref=yb_ref.at[t], dst_ref=ybr_ref.at[my],
                        send_sem=ch_send.at[d], recv_sem=ch_recv.at[my],
                        device_id=(t,),
                        device_id_type=pl.DeviceIdType.MESH,
                    ).start()
                else:
                    pltpu.make_async_copy(
                        yb_ref.at[my], ybr_ref.at[my], own_sem).start()

        amax_ref[my] = jnp.full((8, 128), local_amax, jnp.float32)
        for d in range(1, N_DEV):
            t = lax.rem(my + d, N_DEV)
            pltpu.make_async_remote_copy(
                src_ref=amax_ref.at[my], dst_ref=amax_ref.at[my],
                send_sem=amax_send.at[d], recv_sem=amax_recv.at[my],
                device_id=(t,), device_id_type=pl.DeviceIdType.MESH,
            ).start()
        for d in range(1, N_DEV):
            s = lax.rem(my + d, N_DEV)
            pltpu.make_async_remote_copy(
                src_ref=amax_ref.at[my], dst_ref=amax_ref.at[s],
                send_sem=amax_send.at[d], recv_sem=amax_recv.at[s],
                device_id=(s,), device_id_type=pl.DeviceIdType.MESH,
            ).wait_recv()
        for d in range(1, N_DEV):
            pltpu.make_async_remote_copy(
                src_ref=amax_ref.at[my], dst_ref=amax_ref.at[my],
                send_sem=amax_send.at[d], recv_sem=amax_recv.at[my],
                device_id=(my,), device_id_type=pl.DeviceIdType.MESH,
            ).wait_send()

        gmax = jnp.max(amax_ref[:, :, :])
        scale = gmax / 127.0
        inv_scale = 127.0 / gmax

        inv_b = inv_scale.astype(jnp.bfloat16)
        sc_b = scale.astype(jnp.bfloat16)

        def qdq(s):
            out_ref[pl.ds(s * m_per, m_per), :] = jnp.clip(
                jnp.round(ybr_ref[s] * inv_b), -127.0, 127.0) * sc_b

        pltpu.make_async_copy(
            yb_ref.at[my], ybr_ref.at[my], own_sem).wait()
        qdq(my)
        for d in range(1, N_DEV):
            s = lax.rem(my + d, N_DEV)
            pltpu.make_async_remote_copy(
                src_ref=yb_ref.at[0], dst_ref=ybr_ref.at[s],
                send_sem=ch_send.at[d], recv_sem=ch_recv.at[s],
                device_id=(s,), device_id_type=pl.DeviceIdType.MESH,
            ).wait_recv()
            qdq(s)
        for d in range(1, N_DEV):
            pltpu.make_async_remote_copy(
                src_ref=yb_ref.at[0], dst_ref=ybr_ref.at[my],
                send_sem=ch_send.at[d], recv_sem=ch_recv.at[my],
                device_id=(my,), device_id_type=pl.DeviceIdType.MESH,
            ).wait_send()

    return pl.pallas_call(
        body,
        out_shape=jax.ShapeDtypeStruct((N_DEV * m_per, n_per), jnp.bfloat16),
        in_specs=[
            pl.BlockSpec(memory_space=pl.ANY),
            pl.BlockSpec(memory_space=pl.ANY),
        ],
        out_specs=pl.BlockSpec(memory_space=pltpu.VMEM),
        scratch_shapes=[
            pltpu.VMEM((m_per, X_CHUNK), jnp.float32),
            pltpu.VMEM((m_per, k), jnp.bfloat16),
            pltpu.VMEM((3, k, W_CHUNK), jnp.float32),
            pltpu.VMEM((N_DEV, m_per, n_per), jnp.bfloat16),
            pltpu.VMEM((N_DEV, m_per, n_per), jnp.bfloat16),
            pltpu.VMEM((N_DEV, 8, 128), jnp.float32),
            pltpu.SemaphoreType.DMA,
            pltpu.SemaphoreType.DMA((3,)),
            pltpu.SemaphoreType.DMA,
            pltpu.SemaphoreType.DMA((N_DEV,)),
            pltpu.SemaphoreType.DMA((N_DEV,)),
            pltpu.SemaphoreType.DMA((N_DEV,)),
            pltpu.SemaphoreType.DMA((N_DEV,)),
        ],
        compiler_params=pltpu.CompilerParams(
            vmem_limit_bytes=58 * 1024 * 1024),
    )(x, w_mat)
